# baseline (speedup 1.0000x reference)
import functools
import math

import jax
import jax.numpy as jnp
from jax import lax
from jax.experimental import pallas as pl
from jax.experimental.pallas import tpu as pltpu
from jax.experimental.pallas import tpu_sc as plsc

_PADDING_IDX = 1
_SEQ_BLOCK = 1024
_STRIDE = 16
_L = 16
_C = 32
_NBUF = 2


def _tc_body(x_ref, o_ref):
    S = o_ref.shape[1]
    half = o_ref.shape[2] // 2
    nb = o_ref.shape[0]
    scale = math.log(10000.0) / (half - 1)
    base = (pl.program_id(0) * S + _PADDING_IDX + 1).astype(jnp.float32)

    cols = jax.lax.broadcasted_iota(jnp.int32, (_STRIDE, half), 1)
    freq = jnp.exp(cols.astype(jnp.float32) * (-scale))
    rot_c = jnp.cos(freq * float(_STRIDE))
    rot_s = jnp.sin(freq * float(_STRIDE))

    rows0 = jax.lax.broadcasted_iota(jnp.int32, (_STRIDE, half), 0)
    ang0 = (rows0.astype(jnp.float32) + base) * freq
    sin0 = jnp.sin(ang0)
    cos0 = jnp.cos(ang0)

    def step(k, carry):
        s_k, c_k = carry
        xs = x_ref[pl.ds(k * _STRIDE, _STRIDE), :]
        for b in range(nb):
            m = (xs[:, b:b + 1] != _PADDING_IDX)
            o_ref[b, pl.ds(k * _STRIDE, _STRIDE), :half] = jnp.where(m, s_k, 0.0)
            o_ref[b, pl.ds(k * _STRIDE, _STRIDE), half:] = jnp.where(m, c_k, 0.0)
        s_n = s_k * rot_c + c_k * rot_s
        c_n = c_k * rot_c - s_k * rot_s
        return (s_n, c_n)

    jax.lax.fori_loop(0, S // _STRIDE, step, (sin0, cos0))


def _tc_generate(x, embed_dim, out_dtype):
    bsz, seq_len = x.shape
    S = _SEQ_BLOCK
    num_seq = seq_len // S
    xt = x.T
    return pl.pallas_call(
        _tc_body,
        grid=(num_seq,),
        in_specs=[pl.BlockSpec((S, bsz), lambda i: (i, 0))],
        out_specs=pl.BlockSpec((bsz, S, embed_dim), lambda i: (0, i, 0)),
        out_shape=jax.ShapeDtypeStruct((bsz, seq_len, embed_dim), out_dtype),
    )(xt)


def _sc_lookup(total_rows, seq_len, embed_dim):
    nw = 32
    rows_w = total_rows // nw
    nchunk = rows_w // _C
    ngroups = rows_w // _L
    idx_pad = rows_w + _NBUF * _C

    mesh = plsc.VectorSubcoreMesh(core_axis_name="c", subcore_axis_name="s")

    @functools.partial(
        pl.kernel, mesh=mesh,
        out_type=jax.ShapeDtypeStruct((total_rows, embed_dim), jnp.float32),
        scratch_types=[
            pltpu.VMEM((idx_pad,), jnp.int32),
            pltpu.VMEM((rows_w,), jnp.int32),
            pltpu.VMEM((_NBUF, _C, embed_dim), jnp.float32),
            pltpu.SemaphoreType.DMA,
            pltpu.SemaphoreType.DMA,
        ],
    )
    def k(table_hbm, x_hbm, out_hbm, idx_v, x_v, rows_v, sem0, sem1):
        sems = (sem0, sem1)
        wid = lax.axis_index("s") * 2 + lax.axis_index("c")
        row0 = wid * rows_w
        s0 = lax.rem(row0, seq_len)
        pltpu.sync_copy(x_hbm.at[pl.ds(row0, rows_w)], x_v)

        lane = lax.iota(jnp.int32, _L)

        def mk_idx(g, _):
            xv = x_v[pl.ds(g * _L, _L)]
            pos = (s0 + _PADDING_IDX + 1 + g * _L) + lane
            idx_v[pl.ds(g * _L, _L)] = jnp.where(
                xv == _PADDING_IDX, _PADDING_IDX, pos)
            return 0

        lax.fori_loop(0, ngroups, mk_idx, 0)
        zero = jnp.zeros((_L,), jnp.int32)
        for g in range(ngroups, idx_pad // _L):
            idx_v[pl.ds(g * _L, _L)] = zero

        def gather(kk, b):
            return pltpu.make_async_copy(
                table_hbm.at[idx_v.at[pl.ds(kk * _C, _C)]],
                rows_v.at[b], sems[b])

        for b in range(_NBUF):
            gather(b, b).start()

        def step(g, _):
            for b in range(_NBUF):
                kk = g * _NBUF + b
                gather(kk, b).wait()
                pltpu.sync_copy(rows_v.at[b],
                                out_hbm.at[pl.ds(row0 + kk * _C, _C)])
                gather(kk + _NBUF, b).start()
            return 0

        lax.fori_loop(0, nchunk // _NBUF, step, 0)
        for b in range(_NBUF):
            gather(nchunk + b, b).wait()

    return k


def kernel(x, weights):
    bsz, seq_len = x.shape
    embed_dim = weights.shape[1]
    n_sc = 1
    tc_part = _tc_generate(x[:bsz - n_sc], embed_dim, weights.dtype)
    sc_flat = _sc_lookup(n_sc * seq_len, seq_len, embed_dim)(
        weights, x[bsz - n_sc:].reshape(-1))
    sc_part = sc_flat.reshape(n_sc, seq_len, embed_dim)
    out = jnp.concatenate([tc_part, sc_part], axis=0)
    return jax.lax.stop_gradient(out)


# speedup vs baseline: 4.2411x; 4.2411x over previous
import math

import jax
import jax.numpy as jnp
from jax.experimental import pallas as pl
from jax.experimental.pallas import tpu as pltpu

_PADDING_IDX = 1
_SEQ_BLOCK = 512
_STRIDE = 16
_NQ = 4


def _body(x_ref, o_hbm, buf, sem):
    S = buf.shape[2]
    half = buf.shape[3] // 2
    nb = buf.shape[1]
    scale = math.log(10000.0) / (half - 1)
    i = pl.program_id(0)
    n = pl.num_programs(0)
    q = jax.lax.rem(i, _NQ)
    base = (i * S + _PADDING_IDX + 1).astype(jnp.float32)

    @pl.when(i >= _NQ)
    def _():
        pltpu.make_async_copy(
            buf.at[q], o_hbm.at[:, pl.ds((i - _NQ) * S, S), :], sem.at[q]
        ).wait()

    cols = jax.lax.broadcasted_iota(jnp.int32, (_STRIDE, half), 1)
    freq = jnp.exp(cols.astype(jnp.float32) * (-scale))
    rot_c = jnp.cos(freq * float(_STRIDE))
    rot_s = jnp.sin(freq * float(_STRIDE))

    rows0 = jax.lax.broadcasted_iota(jnp.int32, (_STRIDE, half), 0)
    ang0 = (rows0.astype(jnp.float32) + base) * freq
    sin0 = jnp.sin(ang0)
    cos0 = jnp.cos(ang0)

    def step(k, carry):
        s_k, c_k = carry
        xs = x_ref[pl.ds(k * _STRIDE, _STRIDE), :]
        for b in range(nb):
            m = (xs[:, b:b + 1] != _PADDING_IDX)
            buf[q, b, pl.ds(k * _STRIDE, _STRIDE), :half] = jnp.where(m, s_k, 0.0)
            buf[q, b, pl.ds(k * _STRIDE, _STRIDE), half:] = jnp.where(m, c_k, 0.0)
        s_n = s_k * rot_c + c_k * rot_s
        c_n = c_k * rot_c - s_k * rot_s
        return (s_n, c_n)

    jax.lax.fori_loop(0, S // _STRIDE, step, (sin0, cos0))

    pltpu.make_async_copy(
        buf.at[q], o_hbm.at[:, pl.ds(i * S, S), :], sem.at[q]).start()

    @pl.when(i == n - 1)
    def _():
        for qq in range(_NQ):
            kk = n - _NQ + qq
            pltpu.make_async_copy(
                buf.at[kk % _NQ], o_hbm.at[:, pl.ds(kk * S, S), :],
                sem.at[kk % _NQ]).wait()


def kernel(x, weights):
    bsz, seq_len = x.shape
    embed_dim = weights.shape[1]
    S = _SEQ_BLOCK
    num_seq = seq_len // S
    xt = x.T
    out = pl.pallas_call(
        _body,
        grid=(num_seq,),
        in_specs=[pl.BlockSpec((S, bsz), lambda i: (i, 0))],
        out_specs=pl.BlockSpec(memory_space=pl.ANY),
        out_shape=jax.ShapeDtypeStruct((bsz, seq_len, embed_dim), weights.dtype),
        scratch_shapes=[
            pltpu.VMEM((_NQ, bsz, S, embed_dim), jnp.float32),
            pltpu.SemaphoreType.DMA((_NQ,)),
        ],
    )(xt)
    return jax.lax.stop_gradient(out)
